# baseline (speedup 1.0000x reference)
import functools

import jax
import jax.numpy as jnp
from jax import lax
from jax.experimental import pallas as pl
from jax.experimental.pallas import tpu as pltpu
from jax.experimental.pallas import tpu_sc as plsc

_B, _H, _W, _C = 8, 224, 224, 96
_G = 64
_P = _B * _G * _G
_NW = 32
_KPT = 16
_L = 16

_mesh = plsc.VectorSubcoreMesh(core_axis_name="c", subcore_axis_name="s")


@functools.partial(
    pl.kernel,
    out_type=jax.ShapeDtypeStruct((_P, _C), jnp.float32),
    mesh=_mesh,
    scratch_types=[
        pltpu.VMEM((8, 128), jnp.int32),
        pltpu.VMEM((8, 128), jnp.float32),
        [pltpu.VMEM((2 * _C, _W), jnp.float32) for _ in range(2)],
        [pltpu.VMEM((_G, _C), jnp.float32) for _ in range(2)],
        [pltpu.SemaphoreType.DMA for _ in range(6)],
    ],
    compiler_params=pltpu.CompilerParams(
        needs_layout_passes=False, use_tc_tiling_on_sc=True),
)
def _interp(img, meta_i, meta_f, out, mi, mf, rbufs, obs, sems):
    wid = lax.axis_index("c") * 16 + lax.axis_index("s")
    pltpu.sync_copy(meta_i.at[wid], mi)
    pltpu.sync_copy(meta_f.at[wid], mf)

    iota = lax.iota(jnp.int32, _L)
    r0v = mi[0, pl.ds(0, _L)]
    r1v = mi[0, pl.ds(_L, _L)]
    nrw = mi[0, pl.ds(2 * _L, _L)][0]

    def pair(k, buf, win):
        rows = (r0v[k], r1v[k])
        sms = (sems[2 * (k % 2)], sems[2 * (k % 2) + 1])
        dst = (buf.at[pl.ds(0, _C)], buf.at[pl.ds(_C, _C)])
        if win == 1:
            return tuple(
                (img.at[r, :, pl.ds(0, 128)], d.at[:, pl.ds(0, 128)], sm)
                for r, d, sm in zip(rows, dst, sms))
        return tuple(zip((img.at[r0v[k]], img.at[r1v[k]]), dst, sms))

    def branch2(k, buf, go):
        def mk(win):
            def _f():
                for s, d, sm in pair(k, buf, win):
                    go(s, d, sm)
            return _f

        lax.cond(nrw == 1, mk(1), mk(0))

    def fetch(k, buf):
        branch2(k, buf, lambda s, d, sm: pltpu.async_copy(s, d, sm))

    def waitf(k, buf):
        branch2(k, buf, lambda s, d, sm: pltpu.make_async_copy(s, d, sm).wait())

    fetch(0, rbufs[0])
    wb = [None, None]
    for k in range(_KPT):
        waitf(k, rbufs[k % 2])
        if k + 1 < _KPT:
            fetch(k + 1, rbufs[(k + 1) % 2])
        rb = rbufs[k % 2]
        ob = obs[k % 2]
        if wb[k % 2] is not None:
            wb[k % 2].wait()
        fhs = jnp.full((_L,), mf[0, pl.ds(0, _L)][k])
        ghs = 1.0 - fhs
        qw0 = [mi[1, pl.ds(q * _L, _L)] for q in range(4)]
        qw1 = [mi[2, pl.ds(q * _L, _L)] for q in range(4)]
        qfw = [mf[1, pl.ds(q * _L, _L)] for q in range(4)]
        qa = []
        for q in range(4):
            fw = qfw[q]
            gw = 1.0 - fw
            qa.append((ghs * gw, ghs * fw, fhs * gw, fhs * fw))
        qj = [q * _L + iota for q in range(4)]

        @plsc.parallel_loop(0, _C, 1, unroll=4)
        def cbody(c):
            cs = jnp.full((_L,), c, jnp.int32)
            cs1 = cs + _C
            for q in range(4):
                a00, a01, a10, a11 = qa[q]
                v = plsc.load_gather(rb, [cs, qw0[q]]) * a00
                v = v + plsc.load_gather(rb, [cs, qw1[q]]) * a01
                v = v + plsc.load_gather(rb, [cs1, qw0[q]]) * a10
                v = v + plsc.load_gather(rb, [cs1, qw1[q]]) * a11
                plsc.store_scatter(ob, [qj[q], cs], v)

        wb[k % 2] = pltpu.async_copy(
            ob, out.at[pl.ds((wid * _KPT + k) * _G, _G)], sems[4 + k % 2])
    wb[0].wait()
    wb[1].wait()


def kernel(image, section):
    starts = section[:, :2]
    stops = starts + section[:, 2:3]
    qh = jnp.linspace(starts[:, 0], stops[:, 0], _G, axis=1) * (_H - 1)
    a = jnp.linspace(0.0, 1.0, _G)
    coord2 = (1.0 - a)[None, :] * starts[:, 1][:, None] + a[None, :] * stops[:, 1][:, None]
    qw = coord2 * (_W - 1)
    qh = jnp.clip(qh, 0.0, float(_H - 1))
    qw = jnp.clip(qw, 0.0, float(_W - 1))
    h0 = jnp.floor(qh)
    w0 = jnp.floor(qw)
    h0i = h0.astype(jnp.int32)
    w0i = w0.astype(jnp.int32)
    h1i = jnp.minimum(h0i + 1, _H - 1)
    w1i = jnp.minimum(w0i + 1, _W - 1)
    fh = qh - h0
    fw = qw - w0

    b = jnp.arange(_B, dtype=jnp.int32)[:, None]
    r0 = b * _H + h0i
    r1 = b * _H + h1i

    wmax = jnp.max(w1i, axis=1)
    nrw = (wmax <= 127).astype(jnp.int32)

    r0t = r0.reshape(_NW, _KPT)
    r1t = r1.reshape(_NW, _KPT)
    fht = fh.reshape(_NW, _KPT)
    w0t = jnp.repeat(w0i, 4, axis=0)
    w1t = jnp.repeat(w1i, 4, axis=0)
    fwt = jnp.repeat(fw, 4, axis=0)

    zi = jnp.zeros((_NW, 8, 128), jnp.int32)
    meta_i = zi.at[:, 0, 0:16].set(r0t).at[:, 0, 16:32].set(r1t)
    meta_i = meta_i.at[:, 1, 0:64].set(w0t).at[:, 2, 0:64].set(w1t)
    meta_i = meta_i.at[:, 0, 32].set(jnp.repeat(nrw, 4))
    zf = jnp.zeros((_NW, 8, 128), jnp.float32)
    meta_f = zf.at[:, 0, 0:16].set(fht).at[:, 1, 0:64].set(fwt)

    img = image.transpose(0, 1, 3, 2).reshape(_B * _H, _C, _W)
    out = _interp(img, meta_i, meta_f)
    return out.reshape(_B, _G, _G, _C)


# speedup vs baseline: 1.0677x; 1.0677x over previous
import functools

import jax
import jax.numpy as jnp
from jax import lax
from jax.experimental import pallas as pl
from jax.experimental.pallas import tpu as pltpu
from jax.experimental.pallas import tpu_sc as plsc

_B, _H, _W, _C = 8, 224, 224, 96
_G = 64
_P = _B * _G * _G
_NW = 32
_KPT = 16
_L = 16

_mesh = plsc.VectorSubcoreMesh(core_axis_name="c", subcore_axis_name="s")


@functools.partial(
    pl.kernel,
    out_type=jax.ShapeDtypeStruct((_P, _C), jnp.float32),
    mesh=_mesh,
    scratch_types=[
        pltpu.VMEM((8, 128), jnp.int32),
        pltpu.VMEM((8, 128), jnp.float32),
        [pltpu.VMEM((2 * _C, _W), jnp.float32) for _ in range(2)],
        [pltpu.VMEM((_G, _C), jnp.float32) for _ in range(2)],
        [pltpu.SemaphoreType.DMA for _ in range(6)],
    ],
    compiler_params=pltpu.CompilerParams(
        needs_layout_passes=False, use_tc_tiling_on_sc=True),
)
def _interp(img, meta_i, meta_f, out, mi, mf, rbufs, obs, sems):
    wid = lax.axis_index("c") * 16 + lax.axis_index("s")
    pltpu.sync_copy(meta_i.at[wid], mi)
    pltpu.sync_copy(meta_f.at[wid], mf)

    iota = lax.iota(jnp.int32, _L)
    r0v = mi[0, pl.ds(0, _L)]
    r1v = mi[0, pl.ds(_L, _L)]

    def fetch(k, buf):
        c0 = pltpu.async_copy(img.at[r0v[k]], buf.at[pl.ds(0, _C)],
                              sems[2 * (k % 2)])
        c1 = pltpu.async_copy(img.at[r1v[k]], buf.at[pl.ds(_C, _C)],
                              sems[2 * (k % 2) + 1])
        return (c0, c1)

    cp = fetch(0, rbufs[0])
    wb = [None, None]
    for k in range(_KPT):
        cp[0].wait()
        cp[1].wait()
        if k + 1 < _KPT:
            cp = fetch(k + 1, rbufs[(k + 1) % 2])
        rb = rbufs[k % 2]
        ob = obs[k % 2]
        if wb[k % 2] is not None:
            wb[k % 2].wait()
        fhs = jnp.full((_L,), mf[0, pl.ds(0, _L)][k])
        ghs = 1.0 - fhs
        qw0 = [mi[1, pl.ds(q * _L, _L)] for q in range(4)]
        qw1 = [mi[2, pl.ds(q * _L, _L)] for q in range(4)]
        qfw = [mf[1, pl.ds(q * _L, _L)] for q in range(4)]
        qa = []
        for q in range(4):
            fw = qfw[q]
            gw = 1.0 - fw
            qa.append((ghs * gw, ghs * fw, fhs * gw, fhs * fw))
        qj = [q * _L + iota for q in range(4)]

        @plsc.parallel_loop(0, _C, 1, unroll=4)
        def cbody(c):
            cs = jnp.full((_L,), c, jnp.int32)
            cs1 = cs + _C
            for q in range(4):
                a00, a01, a10, a11 = qa[q]
                v = plsc.load_gather(rb, [cs, qw0[q]]) * a00
                v = v + plsc.load_gather(rb, [cs, qw1[q]]) * a01
                v = v + plsc.load_gather(rb, [cs1, qw0[q]]) * a10
                v = v + plsc.load_gather(rb, [cs1, qw1[q]]) * a11
                plsc.store_scatter(ob, [qj[q], cs], v)

        wb[k % 2] = pltpu.async_copy(
            ob, out.at[pl.ds((wid * _KPT + k) * _G, _G)], sems[4 + k % 2])
    wb[0].wait()
    wb[1].wait()


def kernel(image, section):
    starts = section[:, :2]
    stops = starts + section[:, 2:3]
    qh = jnp.linspace(starts[:, 0], stops[:, 0], _G, axis=1) * (_H - 1)
    a = jnp.linspace(0.0, 1.0, _G)
    coord2 = (1.0 - a)[None, :] * starts[:, 1][:, None] + a[None, :] * stops[:, 1][:, None]
    qw = coord2 * (_W - 1)
    qh = jnp.clip(qh, 0.0, float(_H - 1))
    qw = jnp.clip(qw, 0.0, float(_W - 1))
    h0 = jnp.floor(qh)
    w0 = jnp.floor(qw)
    h0i = h0.astype(jnp.int32)
    w0i = w0.astype(jnp.int32)
    h1i = jnp.minimum(h0i + 1, _H - 1)
    w1i = jnp.minimum(w0i + 1, _W - 1)
    fh = qh - h0
    fw = qw - w0

    b = jnp.arange(_B, dtype=jnp.int32)[:, None]
    r0 = b * _H + h0i
    r1 = b * _H + h1i

    r0t = r0.reshape(_NW, _KPT)
    r1t = r1.reshape(_NW, _KPT)
    fht = fh.reshape(_NW, _KPT)
    w0t = jnp.repeat(w0i, 4, axis=0)
    w1t = jnp.repeat(w1i, 4, axis=0)
    fwt = jnp.repeat(fw, 4, axis=0)

    zi = jnp.zeros((_NW, 8, 128), jnp.int32)
    meta_i = zi.at[:, 0, 0:16].set(r0t).at[:, 0, 16:32].set(r1t)
    meta_i = meta_i.at[:, 1, 0:64].set(w0t).at[:, 2, 0:64].set(w1t)
    zf = jnp.zeros((_NW, 8, 128), jnp.float32)
    meta_f = zf.at[:, 0, 0:16].set(fht).at[:, 1, 0:64].set(fwt)

    img = image.transpose(0, 1, 3, 2).reshape(_B * _H, _C, _W)
    out = _interp(img, meta_i, meta_f)
    return out.reshape(_B, _G, _G, _C)
